# baseline (speedup 1.0000x reference)
import functools

import jax
import jax.numpy as jnp
from jax import lax
from jax.experimental import pallas as pl
from jax.experimental.pallas import tpu as pltpu
from jax.experimental.pallas import tpu_sc as plsc

D = 16
B = 262144
NC = 2
NS = 16
L = 16
NW = NC * NS
BPW = B // NW
C = 2048
NCHUNK = BPW // C

S1 = 129
S2 = 129 * 129


def kernel(idx, table):
    idx_flat = idx.reshape(-1)
    mesh = plsc.VectorSubcoreMesh(core_axis_name="c", subcore_axis_name="s")

    @functools.partial(
        pl.kernel,
        mesh=mesh,
        out_type=jax.ShapeDtypeStruct((B, D), jnp.float32),
        compiler_params=pltpu.CompilerParams(
            needs_layout_passes=False, use_tc_tiling_on_sc=False
        ),
        scratch_types=[
            pltpu.VMEM((3 * C,), jnp.int32),
            pltpu.VMEM((C,), jnp.int32),
            pltpu.VMEM((C, D), jnp.float32),
            pltpu.SemaphoreType.DMA,
        ],
    )
    def grid_gather(idx_hbm, table_hbm, out_hbm, raw_v, lin_v, rows_v, sem):
        wid = lax.axis_index("s") * NC + lax.axis_index("c")
        base_w = wid * BPW

        def chunk_body(ci, carry):
            base = base_w + ci * C
            pltpu.sync_copy(idx_hbm.at[pl.ds(3 * base, 3 * C)], raw_v)

            def lin_body(i, carry2):
                a = 3 * (i * L + lax.iota(jnp.int32, L))
                i0 = plsc.load_gather(raw_v, [a])
                i1 = plsc.load_gather(raw_v, [a + 1])
                i2 = plsc.load_gather(raw_v, [a + 2])
                lin_v[pl.ds(i * L, L)] = i0 + S1 * i1 + S2 * i2
                return carry2

            lax.fori_loop(0, C // L, lin_body, 0)
            pltpu.async_copy(table_hbm.at[lin_v], rows_v, sem).wait()
            pltpu.sync_copy(rows_v, out_hbm.at[pl.ds(base, C)])
            return carry

        lax.fori_loop(0, NCHUNK, chunk_body, 0)

    return grid_gather(idx_flat, table)


# speedup vs baseline: 1.1455x; 1.1455x over previous
import functools

import jax
import jax.numpy as jnp
from jax import lax
from jax.experimental import pallas as pl
from jax.experimental.pallas import tpu as pltpu
from jax.experimental.pallas import tpu_sc as plsc

D = 16
B = 262144
V = 2146689
NC = 2
NS = 16
L = 16
NW = NC * NS
BPW = B // NW
C = 2048
NCHUNK = BPW // C

S1 = 129
S2 = 129 * 129


def kernel(idx, table):
    i0 = idx[:, 0]
    i1 = idx[:, 1]
    i2 = idx[:, 2]
    mesh = plsc.VectorSubcoreMesh(core_axis_name="c", subcore_axis_name="s")

    @functools.partial(
        pl.kernel,
        mesh=mesh,
        out_type=jax.ShapeDtypeStruct((D, B), jnp.float32),
        compiler_params=pltpu.CompilerParams(
            needs_layout_passes=False, use_tc_tiling_on_sc=False
        ),
        scratch_types=[
            pltpu.VMEM((C,), jnp.int32),
            pltpu.VMEM((C,), jnp.int32),
            pltpu.VMEM((C,), jnp.int32),
            pltpu.VMEM((C,), jnp.int32),
            pltpu.VMEM((C, D), jnp.float32),
            pltpu.VMEM((D, C), jnp.float32),
            pltpu.SemaphoreType.DMA,
            pltpu.SemaphoreType.DMA,
        ],
    )
    def grid_gather(i0_hbm, i1_hbm, i2_hbm, tab_hbm, out_hbm,
                    a_v, b_v, c_v, lin_v, rows_v, planes_v, gsem, wsem):
        wid = lax.axis_index("s") * NC + lax.axis_index("c")
        base_w = wid * BPW

        def chunk_body(ci, carry):
            base = base_w + ci * C
            pltpu.sync_copy(i0_hbm.at[pl.ds(base, C)], a_v)
            pltpu.sync_copy(i1_hbm.at[pl.ds(base, C)], b_v)
            pltpu.sync_copy(i2_hbm.at[pl.ds(base, C)], c_v)

            def lin_body(i, carry2):
                s = pl.ds(i * L, L)
                lin_v[s] = a_v[s] + S1 * b_v[s] + S2 * c_v[s]
                return carry2

            lax.fori_loop(0, C // L, lin_body, 0)
            pltpu.async_copy(tab_hbm.at[lin_v], rows_v, gsem).wait()

            lane = lax.iota(jnp.int32, L)

            def tr_body(j, carry2):
                row_idx = j * L + lane
                for f in range(D):
                    col = plsc.load_gather(
                        rows_v, [row_idx, jnp.full((L,), f, jnp.int32)])
                    planes_v[f, pl.ds(j * L, L)] = col
                return carry2

            lax.fori_loop(0, C // L, tr_body, 0)

            wcopies = []
            for f in range(D):
                wcopies.append(pltpu.async_copy(
                    planes_v.at[f], out_hbm.at[f, pl.ds(base, C)], wsem))
            for cp in wcopies:
                cp.wait()
            return carry

        lax.fori_loop(0, NCHUNK, chunk_body, 0)

    out_t = grid_gather(i0, i1, i2, table)
    return out_t.T
